# baseline (speedup 1.0000x reference)
import functools

import jax
import jax.numpy as jnp
from jax import lax
from jax.experimental import pallas as pl
from jax.experimental.pallas import tpu as pltpu
from jax.experimental.pallas import tpu_sc as plsc

VOCAB = 1_000_000
EMB_DIM = 64
NUM_CORES = 2
NUM_SUBCORES = 16
D_PER_CORE = EMB_DIM // NUM_CORES
COL_SLICE = VOCAB // (NUM_SUBCORES // 2)


@functools.lru_cache(maxsize=None)
def _make_kernel(n_fields: int, batch: int):
    b_per_tile = batch // NUM_SUBCORES
    mesh = plsc.VectorSubcoreMesh(core_axis_name="c", subcore_axis_name="s")

    @functools.partial(
        pl.kernel,
        out_type=jax.ShapeDtypeStruct((n_fields, EMB_DIM, batch), jnp.float32),
        mesh=mesh,
        compiler_params=pltpu.CompilerParams(use_tc_tiling_on_sc=False),
        scratch_types=[
            pltpu.VMEM((n_fields * b_per_tile,), jnp.int32),
            pltpu.VMEM((2, n_fields * b_per_tile // 2), jnp.float32),
            pltpu.VMEM_SHARED((VOCAB,), jnp.float32),
            pltpu.SemaphoreType.DMA((2,)),
            pltpu.SemaphoreType.DMA((2,)),
        ],
    )
    def emb_kernel(idx_hbm, wt_hbm, out_hbm, idx_v, res_v, col_s, gsem, wsem):
        c = lax.axis_index("c")
        s = lax.axis_index("s")
        d0 = c * D_PER_CORE
        bbase = s * b_per_tile
        f_half = n_fields // 2
        half = f_half * b_per_tile

        for f in range(n_fields):
            pltpu.sync_copy(
                idx_hbm.at[f, pl.ds(bbase, b_per_tile)],
                idx_v.at[pl.ds(f * b_per_tile, b_per_tile)])

        def wb_issue(d, h):
            for f in range(h * f_half, (h + 1) * f_half):
                pltpu.async_copy(
                    res_v.at[h, pl.ds((f - h * f_half) * b_per_tile,
                                      b_per_tile)],
                    out_hbm.at[f, d, pl.ds(bbase, b_per_tile)],
                    wsem.at[h])

        def wb_drain(h):
            for f in range(f_half):
                pltpu.make_async_copy(
                    res_v.at[h, pl.ds(f * b_per_tile, b_per_tile)],
                    out_hbm.at[f, 0, pl.ds(bbase, b_per_tile)],
                    wsem.at[h]).wait()

        def do_column(d, first):
            @pl.when(s % 2 == 0)
            def _():
                pltpu.sync_copy(
                    wt_hbm.at[d, pl.ds((s // 2) * COL_SLICE, COL_SLICE)],
                    col_s.at[pl.ds((s // 2) * COL_SLICE, COL_SLICE)])
            plsc.subcore_barrier()
            for h in range(2):
                if not first:
                    wb_drain(h)
                pltpu.async_copy(
                    col_s.at[idx_v.at[pl.ds(h * half, half)]],
                    res_v.at[h], gsem.at[h]).wait()
                wb_issue(d, h)
            plsc.subcore_barrier()

        do_column(d0, True)

        def body(p, carry):
            do_column(d0 + p, False)
            return carry

        lax.fori_loop(1, D_PER_CORE, body, 0)

        wb_drain(0)
        wb_drain(1)

    return emb_kernel


def kernel(input, weight):
    batch, n_fields = input.shape
    idx_t = jnp.transpose(input)
    wt_t = jnp.transpose(weight)
    out = _make_kernel(n_fields, batch)(idx_t, wt_t)
    return jnp.transpose(out, (2, 0, 1))


# speedup vs baseline: 4.4529x; 4.4529x over previous
import functools

import jax
import jax.numpy as jnp
from jax import lax
from jax.experimental import pallas as pl
from jax.experimental.pallas import tpu as pltpu
from jax.experimental.pallas import tpu_sc as plsc

V = 1_000_000
D = 64
NUM_CORES = 2
NUM_SUBCORES = 16
NUM_WORKERS = NUM_CORES * NUM_SUBCORES

W1 = 512
NBLK = V // W1
TAIL = V - NBLK * W1
BLK_MAIN = 61


def _mesh():
    return plsc.VectorSubcoreMesh(core_axis_name="c", subcore_axis_name="s")


@functools.lru_cache(maxsize=None)
def _make_transpose_kernel():
    @functools.partial(
        pl.kernel,
        out_type=jax.ShapeDtypeStruct((V * D,), jnp.float32),
        mesh=_mesh(),
        scratch_types=[
            pltpu.VMEM((D, W1), jnp.float32),
            pltpu.VMEM((W1 * D,), jnp.float32),
            pltpu.VMEM((W1 * D,), jnp.float32),
            pltpu.VMEM((D, TAIL), jnp.float32),
            pltpu.SemaphoreType.DMA((2,)),
        ],
    )
    def k1(wt_t, tmp, ibuf, obuf0, obuf1, tailbuf, wsem):
        w = lax.axis_index("s") * NUM_CORES + lax.axis_index("c")
        base = w * BLK_MAIN + jnp.minimum(w, 1)
        iota = lax.iota(jnp.int32, 16)
        row_vecs = [iota + d0 for d0 in range(0, D, 16)]
        obufs = [obuf0, obuf1]

        def transpose_block(src, dst, width):
            def tr(j, carry):
                col = jnp.full((16,), j, jnp.int32)
                for k in range(4):
                    vals = plsc.load_gather(src, [row_vecs[k], col])
                    dst[pl.ds(j * D + k * 16, 16)] = vals
                return carry
            lax.fori_loop(0, width, tr, 0)

        def wb_drain(bb):
            pltpu.make_async_copy(
                obufs[bb], tmp.at[pl.ds(0, W1 * D)], wsem.at[bb]).wait()

        def do_block(q, bb):
            bid = base + q
            valid = jnp.logical_or(q <= BLK_MAIN - 1, w == 0)

            @pl.when(valid)
            def _():
                pltpu.sync_copy(wt_t.at[:, pl.ds(bid * W1, W1)], ibuf)

                @pl.when(q >= 2)
                def _():
                    wb_drain(bb)

                transpose_block(ibuf, obufs[bb], W1)
                pltpu.async_copy(
                    obufs[bb], tmp.at[pl.ds(bid * W1 * D, W1 * D)],
                    wsem.at[bb])

        def pair(pp, carry):
            do_block(2 * pp, 0)
            do_block(2 * pp + 1, 1)
            return carry

        lax.fori_loop(0, 31, pair, 0)
        wb_drain(0)
        wb_drain(1)

        @pl.when(w == 0)
        def _():
            pltpu.sync_copy(wt_t.at[:, pl.ds(NBLK * W1, TAIL)], tailbuf)
            transpose_block(tailbuf, obuf0, TAIL)
            pltpu.sync_copy(
                obuf0.at[pl.ds(0, TAIL * D)],
                tmp.at[pl.ds(NBLK * W1 * D, TAIL * D)])

    return k1


@functools.lru_cache(maxsize=None)
def _make_gather_kernel(n_fields: int, batch: int):
    b_per_w = batch // NUM_WORKERS
    n_bt = b_per_w // 128
    nt = batch // 128

    @functools.partial(
        pl.kernel,
        out_type=jax.ShapeDtypeStruct((n_fields, D // 8, nt, 8 * 128),
                                      jnp.float32),
        mesh=_mesh(),
        compiler_params=pltpu.CompilerParams(use_tc_tiling_on_sc=False, needs_layout_passes=False),
        scratch_types=[
            pltpu.VMEM((n_fields, b_per_w), jnp.int32),
            pltpu.VMEM((n_bt, 128, D), jnp.float32),
            pltpu.VMEM((n_bt, D // 8, 8 * 128), jnp.float32),
            pltpu.SemaphoreType.DMA((n_bt,)),
            pltpu.SemaphoreType.DMA((n_bt,)),
        ],
    )
    def k2(idx, table, out4, ibuf, gbuf, tbuf, gsem, wsem):
        w = lax.axis_index("s") * NUM_CORES + lax.axis_index("c")
        pltpu.sync_copy(idx.at[:, pl.ds(w * b_per_w, b_per_w)], ibuf)
        iota = lax.iota(jnp.int32, 16)
        row_vecs = [iota + jg * 16 for jg in range(8)]

        def gather_issue(f, b):
            pltpu.async_copy(
                table.at[ibuf.at[f, pl.ds(b * 128, 128)]],
                gbuf.at[b], gsem.at[b])

        def gather_wait(b):
            pltpu.make_async_copy(
                table.at[pl.ds(0, 128)], gbuf.at[b], gsem.at[b]).wait()

        def wb_drain(b):
            pltpu.make_async_copy(
                tbuf.at[b], out4.at[0, :, 0, :], wsem.at[b]).wait()

        for b in range(n_bt):
            gather_issue(0, b)

        def group(f, carry):
            for b in range(n_bt):
                gather_wait(b)

                @pl.when(f >= 1)
                def _():
                    wb_drain(b)

                def tr(d, carry2):
                    col = jnp.full((16,), d, jnp.int32)
                    for jg in range(8):
                        vals = plsc.load_gather(
                            gbuf.at[b], [row_vecs[jg], col])
                        tbuf[b, d // 8,
                             pl.ds((d % 8) * 128 + jg * 16, 16)] = vals
                    return carry2
                lax.fori_loop(0, D, tr, 0)

                pltpu.async_copy(
                    tbuf.at[b], out4.at[f, :, w * n_bt + b, :], wsem.at[b])

                @pl.when(f + 1 < n_fields)
                def _():
                    gather_issue(f + 1, b)
            return carry

        lax.fori_loop(0, n_fields, group, 0)
        for b in range(n_bt):
            wb_drain(b)

    return k2


def kernel(input, weight):
    batch, n_fields = input.shape
    idx_t = jnp.transpose(input)
    out4 = _make_gather_kernel(n_fields, batch)(idx_t, weight)
    x = out4.reshape(n_fields, D // 8, batch // 128, 8, 128)
    y = jnp.transpose(x, (2, 4, 0, 1, 3))
    return y.reshape(batch, n_fields, D)


# speedup vs baseline: 6.7378x; 1.5131x over previous
import functools

import jax
import jax.numpy as jnp
from jax import lax
from jax.experimental import pallas as pl
from jax.experimental.pallas import tpu as pltpu
from jax.experimental.pallas import tpu_sc as plsc

V = 1_000_000
D = 64
NUM_CORES = 2
NUM_SUBCORES = 16
NUM_WORKERS = NUM_CORES * NUM_SUBCORES

W1 = 512
NBLK = V // W1
TAIL = V - NBLK * W1
BLK_MAIN = 61


def _mesh():
    return plsc.VectorSubcoreMesh(core_axis_name="c", subcore_axis_name="s")


@functools.lru_cache(maxsize=None)
def _make_transpose_kernel():
    @functools.partial(
        pl.kernel,
        out_type=jax.ShapeDtypeStruct((V * D,), jnp.float32),
        mesh=_mesh(),
        scratch_types=[
            pltpu.VMEM((D, W1), jnp.float32),
            pltpu.VMEM((W1 * D,), jnp.float32),
            pltpu.VMEM((W1 * D,), jnp.float32),
            pltpu.VMEM((D, TAIL), jnp.float32),
            pltpu.SemaphoreType.DMA((2,)),
        ],
    )
    def k1(wt_t, tmp, ibuf, obuf0, obuf1, tailbuf, wsem):
        w = lax.axis_index("s") * NUM_CORES + lax.axis_index("c")
        base = w * BLK_MAIN + jnp.minimum(w, 1)
        iota = lax.iota(jnp.int32, 16)
        row_vecs = [iota + d0 for d0 in range(0, D, 16)]
        obufs = [obuf0, obuf1]

        def transpose_block(src, dst, width):
            def tr(j, carry):
                col = jnp.full((16,), j, jnp.int32)
                for k in range(4):
                    vals = plsc.load_gather(src, [row_vecs[k], col])
                    dst[pl.ds(j * D + k * 16, 16)] = vals
                return carry
            lax.fori_loop(0, width, tr, 0)

        def wb_drain(bb):
            pltpu.make_async_copy(
                obufs[bb], tmp.at[pl.ds(0, W1 * D)], wsem.at[bb]).wait()

        def do_block(q, bb):
            bid = base + q
            valid = jnp.logical_or(q <= BLK_MAIN - 1, w == 0)

            @pl.when(valid)
            def _():
                pltpu.sync_copy(wt_t.at[:, pl.ds(bid * W1, W1)], ibuf)

                @pl.when(q >= 2)
                def _():
                    wb_drain(bb)

                transpose_block(ibuf, obufs[bb], W1)
                pltpu.async_copy(
                    obufs[bb], tmp.at[pl.ds(bid * W1 * D, W1 * D)],
                    wsem.at[bb])

        def pair(pp, carry):
            do_block(2 * pp, 0)
            do_block(2 * pp + 1, 1)
            return carry

        lax.fori_loop(0, 31, pair, 0)
        wb_drain(0)
        wb_drain(1)

        @pl.when(w == 0)
        def _():
            pltpu.sync_copy(wt_t.at[:, pl.ds(NBLK * W1, TAIL)], tailbuf)
            transpose_block(tailbuf, obuf0, TAIL)
            pltpu.sync_copy(
                obuf0.at[pl.ds(0, TAIL * D)],
                tmp.at[pl.ds(NBLK * W1 * D, TAIL * D)])

    return k1


@functools.lru_cache(maxsize=None)
def _make_gather_kernel(n_fields: int, batch: int):
    b_per_w = batch // NUM_WORKERS
    n_bt = b_per_w // 128
    nt = batch // 128

    @functools.partial(
        pl.kernel,
        out_type=jax.ShapeDtypeStruct((n_fields, D // 8, nt, 8, 128),
                                      jnp.float32),
        mesh=_mesh(),
        compiler_params=pltpu.CompilerParams(use_tc_tiling_on_sc=False,
                                             needs_layout_passes=False),
        scratch_types=[
            pltpu.VMEM((n_fields, b_per_w), jnp.int32),
            pltpu.VMEM((n_bt, 128, D), jnp.float32),
            pltpu.VMEM((n_bt, D, 129), jnp.float32),
            pltpu.SemaphoreType.DMA((n_bt,)),
            pltpu.SemaphoreType.DMA((n_bt,)),
        ],
    )
    def k2(idx, table, out5, ibuf, gbuf, tbuf, gsem, wsem):
        w = lax.axis_index("s") * NUM_CORES + lax.axis_index("c")
        pltpu.sync_copy(idx.at[:, pl.ds(w * b_per_w, b_per_w)], ibuf)
        iota = lax.iota(jnp.int32, 16)
        d_vecs = [iota + k * 16 for k in range(4)]

        def gather_issue(f, b):
            pltpu.async_copy(
                table.at[ibuf.at[f, pl.ds(b * 128, 128)]],
                gbuf.at[b], gsem.at[b])

        def gather_wait(b):
            pltpu.make_async_copy(
                table.at[pl.ds(0, 128)], gbuf.at[b], gsem.at[b]).wait()

        def wb_issue(f, b):
            t = w * n_bt + b
            for dg in range(8):
                pltpu.async_copy(
                    tbuf.at[b, pl.ds(dg * 8, 8), pl.ds(0, 128)],
                    out5.at[f, dg, t, :, :], wsem.at[b])

        def wb_drain(b):
            for dg in range(8):
                pltpu.make_async_copy(
                    tbuf.at[b, pl.ds(dg * 8, 8), pl.ds(0, 128)],
                    out5.at[0, 0, 0, :, :], wsem.at[b]).wait()

        for b in range(n_bt):
            gather_issue(0, b)

        def group(f, carry):
            for b in range(n_bt):
                gather_wait(b)

                @pl.when(f >= 1)
                def _():
                    wb_drain(b)

                def tr(j, carry2):
                    col = jnp.full((16,), j, jnp.int32)
                    for k in range(4):
                        vals = gbuf[b, j, pl.ds(k * 16, 16)]
                        plsc.store_scatter(
                            tbuf.at[b], [d_vecs[k], col], vals)
                    return carry2
                lax.fori_loop(0, 128, tr, 0)

                wb_issue(f, b)

                @pl.when(f + 1 < n_fields)
                def _():
                    gather_issue(f + 1, b)
            return carry

        lax.fori_loop(0, n_fields, group, 0)
        for b in range(n_bt):
            wb_drain(b)

    return k2


def kernel(input, weight):
    batch, n_fields = input.shape
    idx_t = jnp.transpose(input)
    out5 = _make_gather_kernel(n_fields, batch)(idx_t, weight)
    y = jnp.transpose(out5, (2, 4, 0, 1, 3))
    return y.reshape(batch, n_fields, D)


# speedup vs baseline: 7.2704x; 1.0790x over previous
import functools

import jax
import jax.numpy as jnp
from jax import lax
from jax.experimental import pallas as pl
from jax.experimental.pallas import tpu as pltpu
from jax.experimental.pallas import tpu_sc as plsc

V = 1_000_000
D = 64
NUM_CORES = 2
NUM_SUBCORES = 16
NUM_WORKERS = NUM_CORES * NUM_SUBCORES

W1 = 512
NBLK = V // W1
TAIL = V - NBLK * W1
BLK_MAIN = 61


def _mesh():
    return plsc.VectorSubcoreMesh(core_axis_name="c", subcore_axis_name="s")


TC_BW = 2048


@functools.lru_cache(maxsize=None)
def _make_tc_detile():
    def body(in_ref, out_ref):
        x = in_ref[...]
        y = jnp.transpose(x)
        y3 = y.reshape(TC_BW // 2, 2, 64)
        out_ref[...] = jnp.concatenate([y3[:, 0, :], y3[:, 1, :]], axis=1)

    def detile(wt_t):
        grid = (V + TC_BW - 1) // TC_BW
        return pl.pallas_call(
            body,
            out_shape=jax.ShapeDtypeStruct((V // 2, 128), jnp.float32),
            grid=(grid,),
            in_specs=[pl.BlockSpec((D, TC_BW), lambda i: (0, i))],
            out_specs=pl.BlockSpec((TC_BW // 2, 128), lambda i: (i, 0)),
        )(wt_t)

    return detile


@functools.lru_cache(maxsize=None)
def _make_transpose_kernel():
    @functools.partial(
        pl.kernel,
        out_type=jax.ShapeDtypeStruct((V * D,), jnp.float32),
        mesh=_mesh(),
        scratch_types=[
            pltpu.VMEM((D, W1), jnp.float32),
            pltpu.VMEM((W1 * D,), jnp.float32),
            pltpu.VMEM((W1 * D,), jnp.float32),
            pltpu.VMEM((D, TAIL), jnp.float32),
            pltpu.SemaphoreType.DMA((2,)),
        ],
    )
    def k1(wt_t, tmp, ibuf, obuf0, obuf1, tailbuf, wsem):
        w = lax.axis_index("s") * NUM_CORES + lax.axis_index("c")
        base = w * BLK_MAIN + jnp.minimum(w, 1)
        iota = lax.iota(jnp.int32, 16)
        row_vecs = [iota + d0 for d0 in range(0, D, 16)]
        obufs = [obuf0, obuf1]

        def transpose_block(src, dst, width):
            def tr(j, carry):
                col = jnp.full((16,), j, jnp.int32)
                for k in range(4):
                    vals = plsc.load_gather(src, [row_vecs[k], col])
                    dst[pl.ds(j * D + k * 16, 16)] = vals
                return carry
            lax.fori_loop(0, width, tr, 0)

        def wb_drain(bb):
            pltpu.make_async_copy(
                obufs[bb], tmp.at[pl.ds(0, W1 * D)], wsem.at[bb]).wait()

        def do_block(q, bb):
            bid = base + q
            valid = jnp.logical_or(q <= BLK_MAIN - 1, w == 0)

            @pl.when(valid)
            def _():
                pltpu.sync_copy(wt_t.at[:, pl.ds(bid * W1, W1)], ibuf)

                @pl.when(q >= 2)
                def _():
                    wb_drain(bb)

                transpose_block(ibuf, obufs[bb], W1)
                pltpu.async_copy(
                    obufs[bb], tmp.at[pl.ds(bid * W1 * D, W1 * D)],
                    wsem.at[bb])

        def pair(pp, carry):
            do_block(2 * pp, 0)
            do_block(2 * pp + 1, 1)
            return carry

        lax.fori_loop(0, 31, pair, 0)
        wb_drain(0)
        wb_drain(1)

        @pl.when(w == 0)
        def _():
            pltpu.sync_copy(wt_t.at[:, pl.ds(NBLK * W1, TAIL)], tailbuf)
            transpose_block(tailbuf, obuf0, TAIL)
            pltpu.sync_copy(
                obuf0.at[pl.ds(0, TAIL * D)],
                tmp.at[pl.ds(NBLK * W1 * D, TAIL * D)])

    return k1


@functools.lru_cache(maxsize=None)
def _make_gather_kernel(n_fields: int, batch: int):
    b_per_w = batch // NUM_WORKERS
    n_bt = b_per_w // 128
    nt = batch // 128

    @functools.partial(
        pl.kernel,
        out_type=jax.ShapeDtypeStruct((n_fields, D // 8, nt, 8, 128),
                                      jnp.float32),
        mesh=_mesh(),
        compiler_params=pltpu.CompilerParams(use_tc_tiling_on_sc=False,
                                             needs_layout_passes=False),
        scratch_types=[
            pltpu.VMEM((n_fields, b_per_w), jnp.int32),
            pltpu.VMEM((n_bt, 128, D), jnp.float32),
            pltpu.VMEM((n_bt, D, 129), jnp.float32),
            pltpu.SemaphoreType.DMA((n_bt,)),
            pltpu.SemaphoreType.DMA((n_bt,)),
        ],
    )
    def k2(idx, table, out5, ibuf, gbuf, tbuf, gsem, wsem):
        w = lax.axis_index("s") * NUM_CORES + lax.axis_index("c")
        pltpu.sync_copy(idx.at[:, pl.ds(w * b_per_w, b_per_w)], ibuf)
        iota = lax.iota(jnp.int32, 16)
        d_vecs = [iota + k * 16 for k in range(4)]

        def gather_issue(f, b):
            pltpu.async_copy(
                table.at[ibuf.at[f, pl.ds(b * 128, 128)]],
                gbuf.at[b], gsem.at[b])

        def gather_wait(b):
            pltpu.make_async_copy(
                table.at[pl.ds(0, 128)], gbuf.at[b], gsem.at[b]).wait()

        def wb_issue(f, b):
            t = w * n_bt + b
            for dg in range(8):
                pltpu.async_copy(
                    tbuf.at[b, pl.ds(dg * 8, 8), pl.ds(0, 128)],
                    out5.at[f, dg, t, :, :], wsem.at[b])

        def wb_drain(b):
            for dg in range(8):
                pltpu.make_async_copy(
                    tbuf.at[b, pl.ds(dg * 8, 8), pl.ds(0, 128)],
                    out5.at[0, 0, 0, :, :], wsem.at[b]).wait()

        for b in range(n_bt):
            gather_issue(0, b)

        def group(f, carry):
            for b in range(n_bt):
                gather_wait(b)

                @pl.when(f >= 1)
                def _():
                    wb_drain(b)

                def tr(j, carry2):
                    col = jnp.full((16,), j, jnp.int32)
                    for k in range(4):
                        vals = gbuf[b, j, pl.ds(k * 16, 16)]
                        plsc.store_scatter(
                            tbuf.at[b], [d_vecs[k], col], vals)
                    return carry2
                lax.fori_loop(0, 128, tr, 0)

                wb_issue(f, b)

                @pl.when(f + 1 < n_fields)
                def _():
                    gather_issue(f + 1, b)
            return carry

        lax.fori_loop(0, n_fields, group, 0)
        for b in range(n_bt):
            wb_drain(b)

    return k2


def kernel(input, weight):
    batch, n_fields = input.shape
    idx_t = jnp.transpose(input)
    wt_t = jnp.transpose(weight)
    table = _make_tc_detile()(wt_t).reshape(V, D)
    out5 = _make_gather_kernel(n_fields, batch)(idx_t, table)
    y = jnp.transpose(out5, (2, 4, 0, 1, 3))
    return y.reshape(batch, n_fields, D)


# speedup vs baseline: 8.5902x; 1.1815x over previous
import functools

import jax
import jax.numpy as jnp
from jax import lax
from jax.experimental import pallas as pl
from jax.experimental.pallas import tpu as pltpu
from jax.experimental.pallas import tpu_sc as plsc

V = 1_000_000
D = 64
NUM_CORES = 2
NUM_SUBCORES = 16
NUM_WORKERS = NUM_CORES * NUM_SUBCORES

W1 = 512
NBLK = V // W1
TAIL = V - NBLK * W1
BLK_MAIN = 61


def _mesh():
    return plsc.VectorSubcoreMesh(core_axis_name="c", subcore_axis_name="s")


TC_BW = 4096


@functools.lru_cache(maxsize=None)
def _make_tc_detile():
    def body(in_ref, out_ref):
        x = in_ref[...]
        y = jnp.transpose(x)
        y3 = y.reshape(TC_BW // 2, 2, 64)
        out_ref[...] = jnp.concatenate([y3[:, 0, :], y3[:, 1, :]], axis=1)

    def detile(wt_t):
        grid = (V + TC_BW - 1) // TC_BW
        return pl.pallas_call(
            body,
            out_shape=jax.ShapeDtypeStruct((V // 2, 128), jnp.float32),
            grid=(grid,),
            in_specs=[pl.BlockSpec((D, TC_BW), lambda i: (0, i))],
            out_specs=pl.BlockSpec((TC_BW // 2, 128), lambda i: (i, 0)),
        )(wt_t)

    return detile


@functools.lru_cache(maxsize=None)
def _make_transpose_kernel():
    @functools.partial(
        pl.kernel,
        out_type=jax.ShapeDtypeStruct((V * D,), jnp.float32),
        mesh=_mesh(),
        scratch_types=[
            pltpu.VMEM((D, W1), jnp.float32),
            pltpu.VMEM((W1 * D,), jnp.float32),
            pltpu.VMEM((W1 * D,), jnp.float32),
            pltpu.VMEM((D, TAIL), jnp.float32),
            pltpu.SemaphoreType.DMA((2,)),
        ],
    )
    def k1(wt_t, tmp, ibuf, obuf0, obuf1, tailbuf, wsem):
        w = lax.axis_index("s") * NUM_CORES + lax.axis_index("c")
        base = w * BLK_MAIN + jnp.minimum(w, 1)
        iota = lax.iota(jnp.int32, 16)
        row_vecs = [iota + d0 for d0 in range(0, D, 16)]
        obufs = [obuf0, obuf1]

        def transpose_block(src, dst, width):
            def tr(j, carry):
                col = jnp.full((16,), j, jnp.int32)
                for k in range(4):
                    vals = plsc.load_gather(src, [row_vecs[k], col])
                    dst[pl.ds(j * D + k * 16, 16)] = vals
                return carry
            lax.fori_loop(0, width, tr, 0)

        def wb_drain(bb):
            pltpu.make_async_copy(
                obufs[bb], tmp.at[pl.ds(0, W1 * D)], wsem.at[bb]).wait()

        def do_block(q, bb):
            bid = base + q
            valid = jnp.logical_or(q <= BLK_MAIN - 1, w == 0)

            @pl.when(valid)
            def _():
                pltpu.sync_copy(wt_t.at[:, pl.ds(bid * W1, W1)], ibuf)

                @pl.when(q >= 2)
                def _():
                    wb_drain(bb)

                transpose_block(ibuf, obufs[bb], W1)
                pltpu.async_copy(
                    obufs[bb], tmp.at[pl.ds(bid * W1 * D, W1 * D)],
                    wsem.at[bb])

        def pair(pp, carry):
            do_block(2 * pp, 0)
            do_block(2 * pp + 1, 1)
            return carry

        lax.fori_loop(0, 31, pair, 0)
        wb_drain(0)
        wb_drain(1)

        @pl.when(w == 0)
        def _():
            pltpu.sync_copy(wt_t.at[:, pl.ds(NBLK * W1, TAIL)], tailbuf)
            transpose_block(tailbuf, obuf0, TAIL)
            pltpu.sync_copy(
                obuf0.at[pl.ds(0, TAIL * D)],
                tmp.at[pl.ds(NBLK * W1 * D, TAIL * D)])

    return k1


@functools.lru_cache(maxsize=None)
def _make_gather_kernel(n_fields: int, batch: int):
    b_per_w = batch // NUM_WORKERS
    n_bt = b_per_w // 128
    nt = batch // 128

    @functools.partial(
        pl.kernel,
        out_type=jax.ShapeDtypeStruct((n_fields, D // 8, nt, 8, 128),
                                      jnp.float32),
        mesh=_mesh(),
        compiler_params=pltpu.CompilerParams(use_tc_tiling_on_sc=False,
                                             needs_layout_passes=False),
        scratch_types=[
            pltpu.VMEM((n_fields, b_per_w), jnp.int32),
            pltpu.VMEM((n_bt, 128, D), jnp.float32),
            pltpu.VMEM((n_bt, D, 129), jnp.float32),
            pltpu.SemaphoreType.DMA((n_bt,)),
            pltpu.SemaphoreType.DMA((n_bt,)),
        ],
    )
    def k2(idx, table, out5, ibuf, gbuf, tbuf, gsem, wsem):
        w = lax.axis_index("s") * NUM_CORES + lax.axis_index("c")
        pltpu.sync_copy(idx.at[:, pl.ds(w * b_per_w, b_per_w)], ibuf)
        iota = lax.iota(jnp.int32, 16)
        d_vecs = [iota + k * 16 for k in range(4)]

        def gather_issue(f, b):
            pltpu.async_copy(
                table.at[ibuf.at[f, pl.ds(b * 128, 128)]],
                gbuf.at[b], gsem.at[b])

        def gather_wait(b):
            pltpu.make_async_copy(
                table.at[pl.ds(0, 128)], gbuf.at[b], gsem.at[b]).wait()

        def wb_issue(f, b):
            t = w * n_bt + b
            for dg in range(8):
                pltpu.async_copy(
                    tbuf.at[b, pl.ds(dg * 8, 8), pl.ds(0, 128)],
                    out5.at[f, dg, t, :, :], wsem.at[b])

        def wb_drain(b):
            for dg in range(8):
                pltpu.make_async_copy(
                    tbuf.at[b, pl.ds(dg * 8, 8), pl.ds(0, 128)],
                    out5.at[0, 0, 0, :, :], wsem.at[b]).wait()

        for b in range(n_bt):
            gather_issue(0, b)

        def group(f, carry):
            for b in range(n_bt):
                gather_wait(b)

                @pl.when(f >= 1)
                def _():
                    wb_drain(b)

                def tr(j, carry2):
                    col = jnp.full((16,), j, jnp.int32)
                    for k in range(4):
                        vals = gbuf[b, j, pl.ds(k * 16, 16)]
                        plsc.store_scatter(
                            tbuf.at[b], [d_vecs[k], col], vals)
                    return carry2
                lax.fori_loop(0, 128, tr, 0)

                wb_issue(f, b)

                @pl.when(f + 1 < n_fields)
                def _():
                    gather_issue(f + 1, b)
            return carry

        lax.fori_loop(0, n_fields, group, 0)
        for b in range(n_bt):
            wb_drain(b)

    return k2


def kernel(input, weight):
    batch, n_fields = input.shape
    idx_t = jnp.transpose(input)
    wt_t = jnp.transpose(weight)
    table = _make_tc_detile()(wt_t).reshape(V, D)
    out5 = _make_gather_kernel(n_fields, batch)(idx_t, table)
    y = jnp.transpose(out5, (2, 4, 0, 1, 3))
    return y.reshape(batch, n_fields, D)


# speedup vs baseline: 8.8715x; 1.0327x over previous
import functools

import jax
import jax.numpy as jnp
from jax import lax
from jax.experimental import pallas as pl
from jax.experimental.pallas import tpu as pltpu
from jax.experimental.pallas import tpu_sc as plsc

V = 1_000_000
D = 64
NUM_CORES = 2
NUM_SUBCORES = 16
NUM_WORKERS = NUM_CORES * NUM_SUBCORES

W1 = 512
NBLK = V // W1
TAIL = V - NBLK * W1
BLK_MAIN = 61


def _mesh():
    return plsc.VectorSubcoreMesh(core_axis_name="c", subcore_axis_name="s")


TC_BW = 8192


@functools.lru_cache(maxsize=None)
def _make_tc_detile():
    def body(in_ref, out_ref):
        x = in_ref[...]
        y = jnp.transpose(x)
        y3 = y.reshape(TC_BW // 2, 2, 64)
        out_ref[...] = jnp.concatenate([y3[:, 0, :], y3[:, 1, :]], axis=1)

    def detile(wt_t):
        grid = (V + TC_BW - 1) // TC_BW
        return pl.pallas_call(
            body,
            out_shape=jax.ShapeDtypeStruct((V // 2, 128), jnp.float32),
            grid=(grid,),
            in_specs=[pl.BlockSpec((D, TC_BW), lambda i: (0, i))],
            out_specs=pl.BlockSpec((TC_BW // 2, 128), lambda i: (i, 0)),
        )(wt_t)

    return detile


@functools.lru_cache(maxsize=None)
def _make_transpose_kernel():
    @functools.partial(
        pl.kernel,
        out_type=jax.ShapeDtypeStruct((V * D,), jnp.float32),
        mesh=_mesh(),
        scratch_types=[
            pltpu.VMEM((D, W1), jnp.float32),
            pltpu.VMEM((W1 * D,), jnp.float32),
            pltpu.VMEM((W1 * D,), jnp.float32),
            pltpu.VMEM((D, TAIL), jnp.float32),
            pltpu.SemaphoreType.DMA((2,)),
        ],
    )
    def k1(wt_t, tmp, ibuf, obuf0, obuf1, tailbuf, wsem):
        w = lax.axis_index("s") * NUM_CORES + lax.axis_index("c")
        base = w * BLK_MAIN + jnp.minimum(w, 1)
        iota = lax.iota(jnp.int32, 16)
        row_vecs = [iota + d0 for d0 in range(0, D, 16)]
        obufs = [obuf0, obuf1]

        def transpose_block(src, dst, width):
            def tr(j, carry):
                col = jnp.full((16,), j, jnp.int32)
                for k in range(4):
                    vals = plsc.load_gather(src, [row_vecs[k], col])
                    dst[pl.ds(j * D + k * 16, 16)] = vals
                return carry
            lax.fori_loop(0, width, tr, 0)

        def wb_drain(bb):
            pltpu.make_async_copy(
                obufs[bb], tmp.at[pl.ds(0, W1 * D)], wsem.at[bb]).wait()

        def do_block(q, bb):
            bid = base + q
            valid = jnp.logical_or(q <= BLK_MAIN - 1, w == 0)

            @pl.when(valid)
            def _():
                pltpu.sync_copy(wt_t.at[:, pl.ds(bid * W1, W1)], ibuf)

                @pl.when(q >= 2)
                def _():
                    wb_drain(bb)

                transpose_block(ibuf, obufs[bb], W1)
                pltpu.async_copy(
                    obufs[bb], tmp.at[pl.ds(bid * W1 * D, W1 * D)],
                    wsem.at[bb])

        def pair(pp, carry):
            do_block(2 * pp, 0)
            do_block(2 * pp + 1, 1)
            return carry

        lax.fori_loop(0, 31, pair, 0)
        wb_drain(0)
        wb_drain(1)

        @pl.when(w == 0)
        def _():
            pltpu.sync_copy(wt_t.at[:, pl.ds(NBLK * W1, TAIL)], tailbuf)
            transpose_block(tailbuf, obuf0, TAIL)
            pltpu.sync_copy(
                obuf0.at[pl.ds(0, TAIL * D)],
                tmp.at[pl.ds(NBLK * W1 * D, TAIL * D)])

    return k1


@functools.lru_cache(maxsize=None)
def _make_gather_kernel(n_fields: int, batch: int):
    b_per_w = batch // NUM_WORKERS
    n_bt = b_per_w // 128
    nt = batch // 128

    @functools.partial(
        pl.kernel,
        out_type=jax.ShapeDtypeStruct((n_fields, D // 8, nt, 8, 128),
                                      jnp.float32),
        mesh=_mesh(),
        compiler_params=pltpu.CompilerParams(use_tc_tiling_on_sc=False,
                                             needs_layout_passes=False),
        scratch_types=[
            pltpu.VMEM((n_fields, b_per_w), jnp.int32),
            pltpu.VMEM((n_bt, 128, D), jnp.float32),
            pltpu.VMEM((n_bt, D, 129), jnp.float32),
            pltpu.SemaphoreType.DMA((n_bt,)),
            pltpu.SemaphoreType.DMA((n_bt,)),
        ],
    )
    def k2(idx, table, out5, ibuf, gbuf, tbuf, gsem, wsem):
        w = lax.axis_index("s") * NUM_CORES + lax.axis_index("c")
        pltpu.sync_copy(idx.at[:, pl.ds(w * b_per_w, b_per_w)], ibuf)
        iota = lax.iota(jnp.int32, 16)
        d_vecs = [iota + k * 16 for k in range(4)]

        def gather_issue(f, b):
            pltpu.async_copy(
                table.at[ibuf.at[f, pl.ds(b * 128, 128)]],
                gbuf.at[b], gsem.at[b])

        def gather_wait(b):
            pltpu.make_async_copy(
                table.at[pl.ds(0, 128)], gbuf.at[b], gsem.at[b]).wait()

        def wb_issue(f, b):
            t = w * n_bt + b
            for dg in range(8):
                pltpu.async_copy(
                    tbuf.at[b, pl.ds(dg * 8, 8), pl.ds(0, 128)],
                    out5.at[f, dg, t, :, :], wsem.at[b])

        def wb_drain(b):
            for dg in range(8):
                pltpu.make_async_copy(
                    tbuf.at[b, pl.ds(dg * 8, 8), pl.ds(0, 128)],
                    out5.at[0, 0, 0, :, :], wsem.at[b]).wait()

        for b in range(n_bt):
            gather_issue(0, b)

        def group(f, carry):
            for b in range(n_bt):
                gather_wait(b)

                @pl.when(f >= 1)
                def _():
                    wb_drain(b)

                def tr(j, carry2):
                    col = jnp.full((16,), j, jnp.int32)
                    for k in range(4):
                        vals = gbuf[b, j, pl.ds(k * 16, 16)]
                        plsc.store_scatter(
                            tbuf.at[b], [d_vecs[k], col], vals)
                    return carry2
                lax.fori_loop(0, 128, tr, 0)

                wb_issue(f, b)

                @pl.when(f + 1 < n_fields)
                def _():
                    gather_issue(f + 1, b)
            return carry

        lax.fori_loop(0, n_fields, group, 0)
        for b in range(n_bt):
            wb_drain(b)

    return k2


def kernel(input, weight):
    batch, n_fields = input.shape
    idx_t = jnp.transpose(input)
    wt_t = jnp.transpose(weight)
    table = _make_tc_detile()(wt_t).reshape(V, D)
    out5 = _make_gather_kernel(n_fields, batch)(idx_t, table)
    y = jnp.transpose(out5, (2, 4, 0, 1, 3))
    return y.reshape(batch, n_fields, D)


# speedup vs baseline: 8.9076x; 1.0041x over previous
import functools

import jax
import jax.numpy as jnp
from jax import lax
from jax.experimental import pallas as pl
from jax.experimental.pallas import tpu as pltpu
from jax.experimental.pallas import tpu_sc as plsc

V = 1_000_000
D = 64
NUM_CORES = 2
NUM_SUBCORES = 16
NUM_WORKERS = NUM_CORES * NUM_SUBCORES

W1 = 512
NBLK = V // W1
TAIL = V - NBLK * W1
BLK_MAIN = 61


def _mesh():
    return plsc.VectorSubcoreMesh(core_axis_name="c", subcore_axis_name="s")


TC_BW = 16384


@functools.lru_cache(maxsize=None)
def _make_tc_detile():
    def body(in_ref, out_ref):
        x = in_ref[...]
        y = jnp.transpose(x)
        y3 = y.reshape(TC_BW // 2, 2, 64)
        out_ref[...] = jnp.concatenate([y3[:, 0, :], y3[:, 1, :]], axis=1)

    def detile(wt_t):
        grid = (V + TC_BW - 1) // TC_BW
        return pl.pallas_call(
            body,
            out_shape=jax.ShapeDtypeStruct((V // 2, 128), jnp.float32),
            grid=(grid,),
            in_specs=[pl.BlockSpec((D, TC_BW), lambda i: (0, i))],
            out_specs=pl.BlockSpec((TC_BW // 2, 128), lambda i: (i, 0)),
        )(wt_t)

    return detile


@functools.lru_cache(maxsize=None)
def _make_transpose_kernel():
    @functools.partial(
        pl.kernel,
        out_type=jax.ShapeDtypeStruct((V * D,), jnp.float32),
        mesh=_mesh(),
        scratch_types=[
            pltpu.VMEM((D, W1), jnp.float32),
            pltpu.VMEM((W1 * D,), jnp.float32),
            pltpu.VMEM((W1 * D,), jnp.float32),
            pltpu.VMEM((D, TAIL), jnp.float32),
            pltpu.SemaphoreType.DMA((2,)),
        ],
    )
    def k1(wt_t, tmp, ibuf, obuf0, obuf1, tailbuf, wsem):
        w = lax.axis_index("s") * NUM_CORES + lax.axis_index("c")
        base = w * BLK_MAIN + jnp.minimum(w, 1)
        iota = lax.iota(jnp.int32, 16)
        row_vecs = [iota + d0 for d0 in range(0, D, 16)]
        obufs = [obuf0, obuf1]

        def transpose_block(src, dst, width):
            def tr(j, carry):
                col = jnp.full((16,), j, jnp.int32)
                for k in range(4):
                    vals = plsc.load_gather(src, [row_vecs[k], col])
                    dst[pl.ds(j * D + k * 16, 16)] = vals
                return carry
            lax.fori_loop(0, width, tr, 0)

        def wb_drain(bb):
            pltpu.make_async_copy(
                obufs[bb], tmp.at[pl.ds(0, W1 * D)], wsem.at[bb]).wait()

        def do_block(q, bb):
            bid = base + q
            valid = jnp.logical_or(q <= BLK_MAIN - 1, w == 0)

            @pl.when(valid)
            def _():
                pltpu.sync_copy(wt_t.at[:, pl.ds(bid * W1, W1)], ibuf)

                @pl.when(q >= 2)
                def _():
                    wb_drain(bb)

                transpose_block(ibuf, obufs[bb], W1)
                pltpu.async_copy(
                    obufs[bb], tmp.at[pl.ds(bid * W1 * D, W1 * D)],
                    wsem.at[bb])

        def pair(pp, carry):
            do_block(2 * pp, 0)
            do_block(2 * pp + 1, 1)
            return carry

        lax.fori_loop(0, 31, pair, 0)
        wb_drain(0)
        wb_drain(1)

        @pl.when(w == 0)
        def _():
            pltpu.sync_copy(wt_t.at[:, pl.ds(NBLK * W1, TAIL)], tailbuf)
            transpose_block(tailbuf, obuf0, TAIL)
            pltpu.sync_copy(
                obuf0.at[pl.ds(0, TAIL * D)],
                tmp.at[pl.ds(NBLK * W1 * D, TAIL * D)])

    return k1


@functools.lru_cache(maxsize=None)
def _make_gather_kernel(n_fields: int, batch: int):
    b_per_w = batch // NUM_WORKERS
    n_bt = b_per_w // 128
    nt = batch // 128

    @functools.partial(
        pl.kernel,
        out_type=jax.ShapeDtypeStruct((n_fields, D // 8, nt, 8, 128),
                                      jnp.float32),
        mesh=_mesh(),
        compiler_params=pltpu.CompilerParams(use_tc_tiling_on_sc=False,
                                             needs_layout_passes=False),
        scratch_types=[
            pltpu.VMEM((n_fields, b_per_w), jnp.int32),
            pltpu.VMEM((n_bt, 128, D), jnp.float32),
            pltpu.VMEM((n_bt, D, 129), jnp.float32),
            pltpu.SemaphoreType.DMA((n_bt,)),
            pltpu.SemaphoreType.DMA((n_bt,)),
        ],
    )
    def k2(idx, table, out5, ibuf, gbuf, tbuf, gsem, wsem):
        w = lax.axis_index("s") * NUM_CORES + lax.axis_index("c")
        pltpu.sync_copy(idx.at[:, pl.ds(w * b_per_w, b_per_w)], ibuf)
        iota = lax.iota(jnp.int32, 16)
        d_vecs = [iota + k * 16 for k in range(4)]

        def gather_issue(f, b):
            pltpu.async_copy(
                table.at[ibuf.at[f, pl.ds(b * 128, 128)]],
                gbuf.at[b], gsem.at[b])

        def gather_wait(b):
            pltpu.make_async_copy(
                table.at[pl.ds(0, 128)], gbuf.at[b], gsem.at[b]).wait()

        def wb_issue(f, b):
            t = w * n_bt + b
            for dg in range(8):
                pltpu.async_copy(
                    tbuf.at[b, pl.ds(dg * 8, 8), pl.ds(0, 128)],
                    out5.at[f, dg, t, :, :], wsem.at[b])

        def wb_drain(b):
            for dg in range(8):
                pltpu.make_async_copy(
                    tbuf.at[b, pl.ds(dg * 8, 8), pl.ds(0, 128)],
                    out5.at[0, 0, 0, :, :], wsem.at[b]).wait()

        for b in range(n_bt):
            gather_issue(0, b)

        def group(f, carry):
            for b in range(n_bt):
                gather_wait(b)

                @pl.when(f >= 1)
                def _():
                    wb_drain(b)

                def tr(j, carry2):
                    col = jnp.full((16,), j, jnp.int32)
                    for k in range(4):
                        vals = gbuf[b, j, pl.ds(k * 16, 16)]
                        plsc.store_scatter(
                            tbuf.at[b], [d_vecs[k], col], vals)
                    return carry2
                lax.fori_loop(0, 128, tr, 0)

                wb_issue(f, b)

                @pl.when(f + 1 < n_fields)
                def _():
                    gather_issue(f + 1, b)
            return carry

        lax.fori_loop(0, n_fields, group, 0)
        for b in range(n_bt):
            wb_drain(b)

    return k2


def kernel(input, weight):
    batch, n_fields = input.shape
    idx_t = jnp.transpose(input)
    wt_t = jnp.transpose(weight)
    table = _make_tc_detile()(wt_t).reshape(V, D)
    out5 = _make_gather_kernel(n_fields, batch)(idx_t, table)
    y = jnp.transpose(out5, (2, 4, 0, 1, 3))
    return y.reshape(batch, n_fields, D)
